# baseline (speedup 1.0000x reference)
import functools

import jax
import jax.numpy as jnp
from jax import lax
from jax.experimental import pallas as pl
from jax.experimental.pallas import tpu as pltpu
from jax.experimental.pallas import tpu_sc as plsc

B, D, E, K = 8192, 2048, 8, 2
H = 2 * D
TM = 512
TN = 512
NJ = H // TN
P = B * K + E * TM
NT = P // TM
LANE = 128

NC, NS = 2, 16
NW = NC * NS


def _gating_body(x_ref, wg_ref, i1_ref, i2_ref, w1_ref, w2_ref):
    logits = jnp.dot(x_ref[...].astype(jnp.bfloat16),
                     wg_ref[...].astype(jnp.bfloat16),
                     preferred_element_type=jnp.float32)
    tb = logits.shape[0]
    iota = lax.broadcasted_iota(jnp.int32, (tb, LANE), 1)
    neg = jnp.float32(-jnp.inf)
    lg = jnp.where(iota < E, logits, neg)
    m1 = jnp.max(lg, axis=1, keepdims=True)
    i1 = jnp.min(jnp.where(lg == m1, iota, LANE), axis=1, keepdims=True)
    l2 = jnp.where(iota == i1, neg, lg)
    m2 = jnp.max(l2, axis=1, keepdims=True)
    i2 = jnp.min(jnp.where(l2 == m2, iota, LANE), axis=1, keepdims=True)
    e = jnp.exp(m2 - m1)
    w1 = 1.0 / (1.0 + e)
    i1_ref[...] = i1
    i2_ref[...] = i2
    w1_ref[...] = w1
    w2_ref[...] = 1.0 - w1


def _gating(x, wg_pad):
    tb = 1024
    outs = (
        jax.ShapeDtypeStruct((B, 1), jnp.int32),
        jax.ShapeDtypeStruct((B, 1), jnp.int32),
        jax.ShapeDtypeStruct((B, 1), jnp.float32),
        jax.ShapeDtypeStruct((B, 1), jnp.float32),
    )
    return pl.pallas_call(
        _gating_body,
        grid=(B // tb,),
        in_specs=[
            pl.BlockSpec((tb, D), lambda m: (m, 0)),
            pl.BlockSpec((D, LANE), lambda m: (0, 0)),
        ],
        out_specs=tuple(pl.BlockSpec((tb, 1), lambda m: (m, 0)) for _ in outs),
        out_shape=outs,
    )(x, wg_pad)


_TN2 = 512
_NJ2 = D // _TN2
_NJT = NJ + _NJ2


def _ffn_body(te_ref, xs_ref, w1_ref, b1_ref, w2_ref, b2_ref, g_ref, out_ref,
              h_ref):
    j = pl.program_id(1)

    @pl.when(j < NJ)
    def _():
        h = jnp.dot(xs_ref[...].astype(jnp.bfloat16),
                    w1_ref[0].astype(jnp.bfloat16),
                    preferred_element_type=jnp.float32)
        h = jnp.maximum(h + b1_ref[0], 0.0)
        h_ref[:, pl.ds(j * TN, TN)] = h.astype(jnp.bfloat16)

    @pl.when(j >= NJ)
    def _():
        y = jnp.dot(h_ref[...], w2_ref[0].astype(jnp.bfloat16),
                    preferred_element_type=jnp.float32)
        out_ref[...] = (y + b2_ref[0]) * g_ref[...]


def _grouped_ffn(te, xs, W1, b1r, W2, b2r, g2):
    grid_spec = pltpu.PrefetchScalarGridSpec(
        num_scalar_prefetch=1,
        grid=(NT, _NJT),
        in_specs=[
            pl.BlockSpec((TM, D), lambda m, j, te: (m, 0)),
            pl.BlockSpec((1, D, TN),
                         lambda m, j, te: (te[m], 0, jnp.minimum(j, NJ - 1))),
            pl.BlockSpec((1, 1, TN),
                         lambda m, j, te: (te[m], 0, jnp.minimum(j, NJ - 1))),
            pl.BlockSpec((1, H, _TN2),
                         lambda m, j, te: (te[m], 0, jnp.maximum(j - NJ, 0))),
            pl.BlockSpec((1, 1, _TN2),
                         lambda m, j, te: (te[m], 0, jnp.maximum(j - NJ, 0))),
            pl.BlockSpec((TM, 1), lambda m, j, te: (m, 0)),
        ],
        out_specs=pl.BlockSpec((TM, _TN2),
                               lambda m, j, te: (m, jnp.maximum(j - NJ, 0))),
        scratch_shapes=[pltpu.VMEM((TM, H), jnp.bfloat16)],
    )
    return pl.pallas_call(
        _ffn_body,
        grid_spec=grid_spec,
        out_shape=jax.ShapeDtypeStruct((P, D), jnp.float32),
        compiler_params=pltpu.CompilerParams(
            dimension_semantics=("arbitrary", "arbitrary")),
    )(te, xs, W1, b1r, W2, b2r, g2)


_GC = 16
_G_ROWS = P // NW
_G_ITERS = _G_ROWS // _GC


@functools.lru_cache(maxsize=None)
def _make_sc_gather():
    mesh = plsc.VectorSubcoreMesh(core_axis_name="c", subcore_axis_name="s")

    @functools.partial(
        pl.kernel,
        mesh=mesh,
        out_type=jax.ShapeDtypeStruct((P, D), jnp.float32),
        scratch_types=[
            pltpu.VMEM((_G_ROWS,), jnp.int32),
            pltpu.VMEM((_GC, D), jnp.float32),
            pltpu.VMEM((_GC, D), jnp.float32),
            pltpu.SemaphoreType.DMA,
            pltpu.SemaphoreType.DMA,
            pltpu.SemaphoreType.DMA,
            pltpu.SemaphoreType.DMA,
        ],
    )
    def _sc_gather(x_hbm, tok_hbm, xs_hbm, idx_v, r0, r1, sg0, sg1, sw0, sw1):
        wid = lax.axis_index("s") * NC + lax.axis_index("c")
        base = wid * _G_ROWS

        def gth(i, buf, sem):
            return pltpu.make_async_copy(
                x_hbm.at[idx_v.at[pl.ds(i * _GC, _GC)]], buf, sem)

        def wrb(i, buf, sem):
            return pltpu.make_async_copy(
                buf, xs_hbm.at[pl.ds(base + i * _GC, _GC)], sem)

        pltpu.sync_copy(tok_hbm.at[pl.ds(base, _G_ROWS)], idx_v)
        gth(0, r0, sg0).start()

        def body(k, carry):
            i0 = 2 * k
            gth(i0, r0, sg0).wait()

            @pl.when(k > 0)
            def _():
                wrb(i0, r1, sw1).wait()
            gth(i0 + 1, r1, sg1).start()
            wrb(i0, r0, sw0).start()
            gth(i0 + 1, r1, sg1).wait()

            @pl.when(k < _G_ITERS // 2 - 1)
            def _():
                wrb(i0, r0, sw0).wait()
                gth(i0 + 2, r0, sg0).start()
            wrb(i0 + 1, r1, sw1).start()
            return carry

        lax.fori_loop(0, _G_ITERS // 2, body, 0)
        wrb(0, r0, sw0).wait()
        wrb(0, r1, sw1).wait()

    return _sc_gather


_CC = 8
_C_ROWS = B // NW
_C_ITERS = _C_ROWS // _CC
_DV = D // 16


@functools.lru_cache(maxsize=None)
def _make_sc_combine():
    mesh = plsc.VectorSubcoreMesh(core_axis_name="c", subcore_axis_name="s")

    @functools.partial(
        pl.kernel,
        mesh=mesh,
        out_type=jax.ShapeDtypeStruct((B, D), jnp.float32),
        scratch_types=[
            pltpu.VMEM((2 * _C_ROWS,), jnp.int32),
            pltpu.VMEM((2 * _CC, D), jnp.float32),
            pltpu.VMEM((2 * _CC, D), jnp.float32),
            pltpu.VMEM((_CC, D), jnp.float32),
            pltpu.VMEM((_CC, D), jnp.float32),
            pltpu.SemaphoreType.DMA,
            pltpu.SemaphoreType.DMA,
            pltpu.SemaphoreType.DMA,
            pltpu.SemaphoreType.DMA,
        ],
    )
    def _sc_combine(y_hbm, dest_hbm, out_hbm, idx_v, p0, p1, r0, r1,
                    sg0, sg1, sw0, sw1):
        wid = lax.axis_index("s") * NC + lax.axis_index("c")
        base = wid * _C_ROWS

        def gth(i, buf, sem):
            return pltpu.make_async_copy(
                y_hbm.at[idx_v.at[pl.ds(2 * i * _CC, 2 * _CC)]], buf, sem)

        def wrb(i, buf, sem):
            return pltpu.make_async_copy(
                buf, out_hbm.at[pl.ds(base + i * _CC, _CC)], sem)

        def addpairs(pbuf, rbuf):
            def add_body(k, c2):
                r = k // _DV
                col = (k % _DV) * 16
                rbuf[r, pl.ds(col, 16)] = (
                    pbuf[2 * r, pl.ds(col, 16)] + pbuf[2 * r + 1, pl.ds(col, 16)])
                return c2
            lax.fori_loop(0, _CC * _DV, add_body, 0)

        pltpu.sync_copy(dest_hbm.at[pl.ds(2 * base, 2 * _C_ROWS)], idx_v)
        gth(0, p0, sg0).start()

        def body(k, carry):
            i0 = 2 * k
            gth(i0, p0, sg0).wait()
            gth(i0 + 1, p1, sg1).start()

            @pl.when(k > 0)
            def _():
                wrb(i0, r0, sw0).wait()
            addpairs(p0, r0)
            wrb(i0, r0, sw0).start()
            gth(i0 + 1, p1, sg1).wait()

            @pl.when(k < _C_ITERS // 2 - 1)
            def _():
                gth(i0 + 2, p0, sg0).start()

            @pl.when(k > 0)
            def _():
                wrb(i0 + 1, r1, sw1).wait()
            addpairs(p1, r1)
            wrb(i0 + 1, r1, sw1).start()
            return carry

        lax.fori_loop(0, _C_ITERS // 2, body, 0)
        wrb(0, r0, sw0).wait()
        wrb(0, r1, sw1).wait()

    return _sc_combine


def kernel(x, Wg, W1, b1, W2, b2):
    wg_pad = jnp.zeros((D, LANE), jnp.float32).at[:, :E].set(Wg)
    i1, i2, w1, w2 = _gating(x, wg_pad)

    eflat = jnp.concatenate([i1, i2], axis=1).reshape(-1)
    wflat = jnp.concatenate([w1, w2], axis=1).reshape(-1)
    lanes = jnp.arange(E, dtype=jnp.int32)
    onehot = (eflat[:, None] == lanes[None, :]).astype(jnp.int32)
    cc = jnp.cumsum(onehot, axis=0)
    counts = cc[-1]
    rank = jnp.take_along_axis(cc, eflat[:, None], axis=1)[:, 0] - 1
    sizes_pad = ((counts + TM - 1) // TM) * TM
    offs_pad = jnp.concatenate(
        [jnp.zeros((1,), jnp.int32), jnp.cumsum(sizes_pad)[:-1]])
    dest = offs_pad[eflat] + rank
    tok = jnp.arange(B * K, dtype=jnp.int32) // K
    row_token = jnp.zeros((P,), jnp.int32).at[dest].set(tok)
    row_gate = jnp.zeros((P,), jnp.float32).at[dest].set(wflat)
    offs_end = jnp.cumsum(sizes_pad)
    te = jnp.minimum(
        jnp.sum((jnp.arange(NT, dtype=jnp.int32)[:, None] * TM)
                >= offs_end[None, :], axis=1).astype(jnp.int32), E - 1)
    xs = _make_sc_gather()(x, row_token)
    yrows = _grouped_ffn(te, xs, W1, b1.reshape(E, 1, H),
                         W2, b2.reshape(E, 1, D), row_gate.reshape(P, 1))
    return _make_sc_combine()(yrows, dest)


# speedup vs baseline: 1.0639x; 1.0639x over previous
import functools

import jax
import jax.numpy as jnp
from jax import lax
from jax.experimental import pallas as pl
from jax.experimental.pallas import tpu as pltpu
from jax.experimental.pallas import tpu_sc as plsc

B, D, E, K = 8192, 2048, 8, 2
H = 2 * D
TM = 512
TN = 512
NJ = H // TN
P = B * K + E * TM
NT = P // TM
LANE = 128

NC, NS = 2, 16
NW = NC * NS


def _gating_body(x_ref, wg_ref, idx_ref, w_ref):
    logits = jnp.dot(x_ref[...].astype(jnp.bfloat16),
                     wg_ref[...].astype(jnp.bfloat16),
                     preferred_element_type=jnp.float32)
    tb = logits.shape[0]
    iota = lax.broadcasted_iota(jnp.int32, (tb, LANE), 1)
    neg = jnp.float32(-jnp.inf)
    lg = jnp.where(iota < E, logits, neg)
    m1 = jnp.max(lg, axis=1, keepdims=True)
    i1 = jnp.min(jnp.where(lg == m1, iota, LANE), axis=1, keepdims=True)
    l2 = jnp.where(iota == i1, neg, lg)
    m2 = jnp.max(l2, axis=1, keepdims=True)
    i2 = jnp.min(jnp.where(l2 == m2, iota, LANE), axis=1, keepdims=True)
    e = jnp.exp(m2 - m1)
    w1 = 1.0 / (1.0 + e)
    idx_ref[...] = jnp.concatenate([i1, i2], axis=1)
    w_ref[...] = jnp.concatenate([w1, 1.0 - w1], axis=1)


def _gating(x, wg_pad):
    tb = 1024
    outs = (
        jax.ShapeDtypeStruct((B, K), jnp.int32),
        jax.ShapeDtypeStruct((B, K), jnp.float32),
    )
    return pl.pallas_call(
        _gating_body,
        grid=(B // tb,),
        in_specs=[
            pl.BlockSpec((tb, D), lambda m: (m, 0)),
            pl.BlockSpec((D, LANE), lambda m: (0, 0)),
        ],
        out_specs=tuple(pl.BlockSpec((tb, K), lambda m: (m, 0)) for _ in outs),
        out_shape=outs,
    )(x, wg_pad)


def _ffn_body(te_ref, xs_ref, w1_ref, b1_ref, w2_ref, b2_ref, g_ref, out_ref):
    j = pl.program_id(1)
    h = jnp.dot(xs_ref[...].astype(jnp.bfloat16), w1_ref[0].astype(jnp.bfloat16),
                preferred_element_type=jnp.float32)
    h = jnp.maximum(h + b1_ref[0], 0.0)
    contrib = jnp.dot(h.astype(jnp.bfloat16), w2_ref[0].astype(jnp.bfloat16),
                      preferred_element_type=jnp.float32)

    @pl.when(j == 0)
    def _():
        out_ref[...] = contrib

    @pl.when(j > 0)
    def _():
        out_ref[...] += contrib

    @pl.when(j == NJ - 1)
    def _():
        out_ref[...] = (out_ref[...] + b2_ref[0]) * g_ref[...]


def _grouped_ffn(te, xs, W1, b1r, W2, b2r, g2):
    grid_spec = pltpu.PrefetchScalarGridSpec(
        num_scalar_prefetch=1,
        grid=(NT, NJ),
        in_specs=[
            pl.BlockSpec((TM, D), lambda m, j, te: (m, 0)),
            pl.BlockSpec((1, D, TN), lambda m, j, te: (te[m], 0, j)),
            pl.BlockSpec((1, 1, TN), lambda m, j, te: (te[m], 0, j)),
            pl.BlockSpec((1, TN, D), lambda m, j, te: (te[m], j, 0)),
            pl.BlockSpec((1, 1, D), lambda m, j, te: (te[m], 0, 0)),
            pl.BlockSpec((TM, 1), lambda m, j, te: (m, 0)),
        ],
        out_specs=pl.BlockSpec((TM, D), lambda m, j, te: (m, 0)),
    )
    return pl.pallas_call(
        _ffn_body,
        grid_spec=grid_spec,
        out_shape=jax.ShapeDtypeStruct((P, D), jnp.float32),
        compiler_params=pltpu.CompilerParams(
            dimension_semantics=("arbitrary", "arbitrary")),
    )(te, xs, W1, b1r, W2, b2r, g2)


_GC = 8
_G_RING = 4
_G_ROWS = P // NW
_G_ITERS = _G_ROWS // _GC


@functools.lru_cache(maxsize=None)
def _make_sc_gather():
    mesh = plsc.VectorSubcoreMesh(core_axis_name="c", subcore_axis_name="s")

    @functools.partial(
        pl.kernel,
        mesh=mesh,
        out_type=jax.ShapeDtypeStruct((P, D), jnp.float32),
        scratch_types=[
            pltpu.VMEM((_G_ROWS,), jnp.int32),
        ] + [pltpu.VMEM((_GC, D), jnp.float32)] * _G_RING
          + [pltpu.SemaphoreType.DMA] * (2 * _G_RING),
    )
    def _sc_gather(x_hbm, tok_hbm, xs_hbm, idx_v, *bufsem):
        rbufs = bufsem[:_G_RING]
        sgs = bufsem[_G_RING:2 * _G_RING]
        sws = bufsem[2 * _G_RING:]
        wid = lax.axis_index("s") * NC + lax.axis_index("c")
        base = wid * _G_ROWS

        def gth(i, b):
            return pltpu.make_async_copy(
                x_hbm.at[idx_v.at[pl.ds(i * _GC, _GC)]], rbufs[b], sgs[b])

        def wrb(i, b):
            return pltpu.make_async_copy(
                rbufs[b], xs_hbm.at[pl.ds(base + i * _GC, _GC)], sws[b])

        pltpu.sync_copy(tok_hbm.at[pl.ds(base, _G_ROWS)], idx_v)
        for b in range(_G_RING - 1):
            gth(b, b).start()

        def body(k, carry):
            i0 = _G_RING * k
            for b in range(_G_RING):
                i = i0 + b
                nb = (b + _G_RING - 1) % _G_RING
                gth(i, b).wait()

                @pl.when(i + _G_RING - 1 < _G_ITERS)
                def _():
                    @pl.when(i > 0)
                    def _():
                        wrb(i - 1, nb).wait()
                    gth(i + _G_RING - 1, nb).start()
                wrb(i, b).start()
            return carry

        lax.fori_loop(0, _G_ITERS // _G_RING, body, 0)
        for b in range(_G_RING):
            wrb(0, b).wait()

    return _sc_gather


_CC = 8
_C_ROWS = B // NW
_C_ITERS = _C_ROWS // _CC
_DV = D // 16


@functools.lru_cache(maxsize=None)
def _make_sc_combine():
    mesh = plsc.VectorSubcoreMesh(core_axis_name="c", subcore_axis_name="s")

    @functools.partial(
        pl.kernel,
        mesh=mesh,
        out_type=jax.ShapeDtypeStruct((B, D), jnp.float32),
        scratch_types=[
            pltpu.VMEM((2 * _C_ROWS,), jnp.int32),
            pltpu.VMEM((2 * _CC, D), jnp.float32),
            pltpu.VMEM((2 * _CC, D), jnp.float32),
            pltpu.VMEM((_CC, D), jnp.float32),
            pltpu.VMEM((_CC, D), jnp.float32),
            pltpu.SemaphoreType.DMA,
            pltpu.SemaphoreType.DMA,
            pltpu.SemaphoreType.DMA,
            pltpu.SemaphoreType.DMA,
        ],
    )
    def _sc_combine(y_hbm, dest_hbm, out_hbm, idx_v, p0, p1, r0, r1,
                    sg0, sg1, sw0, sw1):
        wid = lax.axis_index("s") * NC + lax.axis_index("c")
        base = wid * _C_ROWS

        def gth(i, buf, sem):
            return pltpu.make_async_copy(
                y_hbm.at[idx_v.at[pl.ds(2 * i * _CC, 2 * _CC)]], buf, sem)

        def wrb(i, buf, sem):
            return pltpu.make_async_copy(
                buf, out_hbm.at[pl.ds(base + i * _CC, _CC)], sem)

        def addpairs(pbuf, rbuf):
            def add_body(k, c2):
                r = k // _DV
                col = (k % _DV) * 16
                rbuf[r, pl.ds(col, 16)] = (
                    pbuf[2 * r, pl.ds(col, 16)] + pbuf[2 * r + 1, pl.ds(col, 16)])
                return c2
            lax.fori_loop(0, _CC * _DV, add_body, 0)

        pltpu.sync_copy(dest_hbm.at[pl.ds(2 * base, 2 * _C_ROWS)], idx_v)
        gth(0, p0, sg0).start()

        def body(k, carry):
            i0 = 2 * k
            gth(i0, p0, sg0).wait()
            gth(i0 + 1, p1, sg1).start()

            @pl.when(k > 0)
            def _():
                wrb(i0, r0, sw0).wait()
            addpairs(p0, r0)
            wrb(i0, r0, sw0).start()
            gth(i0 + 1, p1, sg1).wait()

            @pl.when(k < _C_ITERS // 2 - 1)
            def _():
                gth(i0 + 2, p0, sg0).start()

            @pl.when(k > 0)
            def _():
                wrb(i0 + 1, r1, sw1).wait()
            addpairs(p1, r1)
            wrb(i0 + 1, r1, sw1).start()
            return carry

        lax.fori_loop(0, _C_ITERS // 2, body, 0)
        wrb(0, r0, sw0).wait()
        wrb(0, r1, sw1).wait()

    return _sc_combine


def kernel(x, Wg, W1, b1, W2, b2):
    wg_pad = jnp.zeros((D, LANE), jnp.float32).at[:, :E].set(Wg)
    idx2, w2_ = _gating(x, wg_pad)

    eflat = idx2.reshape(-1)
    wflat = w2_.reshape(-1)
    lanes = jnp.arange(E, dtype=jnp.int32)
    onehot = (eflat[:, None] == lanes[None, :]).astype(jnp.int32)
    cc = jnp.cumsum(onehot, axis=0)
    counts = cc[-1]
    rank = jnp.sum(cc * onehot, axis=1) - 1
    sizes_pad = ((counts + TM - 1) // TM) * TM
    offs_pad = jnp.concatenate(
        [jnp.zeros((1,), jnp.int32), jnp.cumsum(sizes_pad)[:-1]])
    dest = jnp.sum(onehot * offs_pad[None, :], axis=1) + rank
    tok = jnp.arange(B * K, dtype=jnp.int32) // K
    row_token = jnp.zeros((P,), jnp.int32).at[dest].set(tok)
    row_gate = jnp.zeros((P,), jnp.float32).at[dest].set(wflat)
    offs_end = jnp.cumsum(sizes_pad)
    te = jnp.minimum(
        jnp.sum((jnp.arange(NT, dtype=jnp.int32)[:, None] * TM)
                >= offs_end[None, :], axis=1).astype(jnp.int32), E - 1)
    xs = _make_sc_gather()(x, row_token)
    yrows = _grouped_ffn(te, xs, W1, b1.reshape(E, 1, H),
                         W2, b2.reshape(E, 1, D), row_gate.reshape(P, 1))
    return _make_sc_combine()(yrows, dest)
